# Initial kernel scaffold; baseline (speedup 1.0000x reference)
import jax
import jax.numpy as jnp
from jax.experimental import pallas as pl


def kernel(x, edge_index, batch, agg_scatter_index_0, agg_scatter_index_1, agg_scatter_index_2, agg_node_index_0, agg_node_index_1, agg_node_index_2, W_pre, eps, mlp_w1, mlp_w2, head_w, head_b):
    raise NotImplementedError("write your pallas kernel here")


# speedup vs baseline: 2.9189x; 2.9189x over previous
import functools

import jax
import jax.numpy as jnp
from jax import lax
from jax.experimental import pallas as pl
from jax.experimental.pallas import tpu as pltpu
from jax.experimental.pallas import tpu_sc as plsc

_N = 10000
_E = 320000
_D = 128
_DOUT = 64
_HOPS = 3

_CHUNK = 128
_NWORK = 32
_CH_PER_W = 79
_EPAD = _NWORK * _CH_PER_W * _CHUNK
_RPT = 624
_RREM = _N - 16 * _RPT
_ACC_ROWS = _N + 16


def _sc_agg_body(cur_hbm, s0_hbm, s1_hbm, s2_hbm, d0_hbm, d1_hbm, d2_hbm,
                 out_hbm, sidx_v, didx_v, rows_v, acc, sem):
    c = lax.axis_index("c")
    s = lax.axis_index("s")
    w = s * 2 + c
    shbm = [s0_hbm, s1_hbm, s2_hbm]
    dhbm = [d0_hbm, d1_hbm, d2_hbm]

    for hop in range(_HOPS):
        pltpu.sync_copy(cur_hbm.at[pl.ds(s * _RPT, _RPT)],
                        acc.at[pl.ds(s * _RPT, _RPT)])

        @pl.when(s == 15)
        def _():
            pltpu.sync_copy(cur_hbm.at[pl.ds(16 * _RPT, _RREM)],
                            acc.at[pl.ds(16 * _RPT, _RREM)])

        plsc.subcore_barrier()

        def body(i, carry):
            off = (w * _CH_PER_W + i) * _CHUNK
            pltpu.sync_copy(shbm[hop].at[pl.ds(off, _CHUNK)], sidx_v)
            pltpu.sync_copy(dhbm[hop].at[pl.ds(off, _CHUNK)], didx_v)
            pltpu.async_copy(cur_hbm.at[sidx_v], rows_v, sem).wait()
            pltpu.sync_copy(rows_v, acc.at[didx_v], add=True)
            return carry

        lax.fori_loop(0, _CH_PER_W, body, 0)
        plsc.subcore_barrier()
        pltpu.sync_copy(acc.at[pl.ds(s * _RPT, _RPT)],
                        out_hbm.at[hop, c, pl.ds(s * _RPT, _RPT)])

        @pl.when(s == 15)
        def _():
            pltpu.sync_copy(acc.at[pl.ds(16 * _RPT, _RREM)],
                            out_hbm.at[hop, c, pl.ds(16 * _RPT, _RREM)])

        plsc.subcore_barrier()


def _sc_agg(cur, sidx, didx):
    mesh = plsc.VectorSubcoreMesh(core_axis_name="c", subcore_axis_name="s")
    return pl.kernel(
        _sc_agg_body,
        out_type=jax.ShapeDtypeStruct((_HOPS, 2, _N, _D), jnp.float32),
        mesh=mesh,
        scratch_types=[
            pltpu.VMEM((_CHUNK,), jnp.int32),
            pltpu.VMEM((_CHUNK,), jnp.int32),
            pltpu.VMEM((_CHUNK, _D), jnp.float32),
            pltpu.VMEM_SHARED((_ACC_ROWS, _D), jnp.float32),
            pltpu.SemaphoreType.DMA,
        ],
    )(cur, sidx[0], sidx[1], sidx[2], didx[0], didx[1], didx[2])


_ROWS_BLK = 1000


def _pre_body(x_ref, w_ref, o_ref):
    o_ref[...] = jnp.maximum(
        jnp.dot(x_ref[...], w_ref[...], preferred_element_type=jnp.float32),
        0.0)


def _tc_pre(x, w_pre):
    return pl.pallas_call(
        _pre_body,
        grid=(_N // _ROWS_BLK,),
        in_specs=[
            pl.BlockSpec((_ROWS_BLK, _D), lambda i: (i, 0)),
            pl.BlockSpec((_D, _D), lambda i: (0, 0)),
        ],
        out_specs=pl.BlockSpec((_ROWS_BLK, _D), lambda i: (i, 0)),
        out_shape=jax.ShapeDtypeStruct((_N, _D), jnp.float32),
    )(x, w_pre)


def _layer_body(eps_ref, cur_ref, p_ref, w1_ref, w2_ref, o_ref):
    cur = cur_ref[...]
    acc = (1.0 + eps_ref[0]) * cur
    for h in range(_HOPS):
        agg = p_ref[h, 0] + p_ref[h, 1] - cur
        t = jnp.maximum(
            jnp.dot(agg, w1_ref[h], preferred_element_type=jnp.float32), 0.0)
        acc = acc + jnp.dot(t, w2_ref[h], preferred_element_type=jnp.float32)
    o_ref[...] = acc


def _tc_layer(cur, parts, w1, w2, eps):
    return pl.pallas_call(
        _layer_body,
        grid=(_N // _ROWS_BLK,),
        in_specs=[
            pl.BlockSpec(memory_space=pltpu.SMEM),
            pl.BlockSpec((_ROWS_BLK, _D), lambda i: (i, 0)),
            pl.BlockSpec((_HOPS, 2, _ROWS_BLK, _D), lambda i: (0, 0, i, 0)),
            pl.BlockSpec((_HOPS, _D, _D), lambda i: (0, 0, 0)),
            pl.BlockSpec((_HOPS, _D, _D), lambda i: (0, 0, 0)),
        ],
        out_specs=pl.BlockSpec((_ROWS_BLK, _D), lambda i: (i, 0)),
        out_shape=jax.ShapeDtypeStruct((_N, _D), jnp.float32),
    )(eps, cur, parts, w1, w2)


def _layer_head_body(eps_ref, cur_ref, p_ref, w1_ref, w2_ref, hw_ref, hb_ref,
                     o_ref):
    cur = cur_ref[...]
    acc = (1.0 + eps_ref[0]) * cur
    for h in range(_HOPS):
        agg = p_ref[h, 0] + p_ref[h, 1] - cur
        t = jnp.maximum(
            jnp.dot(agg, w1_ref[h], preferred_element_type=jnp.float32), 0.0)
        acc = acc + jnp.dot(t, w2_ref[h], preferred_element_type=jnp.float32)
    o_ref[...] = (
        jnp.dot(acc, hw_ref[...], preferred_element_type=jnp.float32)
        + hb_ref[...])


def _tc_layer_head(cur, parts, w1, w2, eps, head_w, head_b):
    return pl.pallas_call(
        _layer_head_body,
        grid=(_N // _ROWS_BLK,),
        in_specs=[
            pl.BlockSpec(memory_space=pltpu.SMEM),
            pl.BlockSpec((_ROWS_BLK, _D), lambda i: (i, 0)),
            pl.BlockSpec((_HOPS, 2, _ROWS_BLK, _D), lambda i: (0, 0, i, 0)),
            pl.BlockSpec((_HOPS, _D, _D), lambda i: (0, 0, 0)),
            pl.BlockSpec((_HOPS, _D, _D), lambda i: (0, 0, 0)),
            pl.BlockSpec((_D, _DOUT), lambda i: (0, 0)),
            pl.BlockSpec((1, _DOUT), lambda i: (0, 0)),
        ],
        out_specs=pl.BlockSpec((_ROWS_BLK, _DOUT), lambda i: (i, 0)),
        out_shape=jax.ShapeDtypeStruct((_N, _DOUT), jnp.float32),
    )(eps, cur, parts, w1, w2, head_w, head_b.reshape(1, _DOUT))


def kernel(x, edge_index, batch,
           agg_scatter_index_0, agg_scatter_index_1, agg_scatter_index_2,
           agg_node_index_0, agg_node_index_1, agg_node_index_2,
           W_pre, eps, mlp_w1, mlp_w2, head_w, head_b):
    sidx = jnp.stack(
        [agg_scatter_index_0, agg_scatter_index_1, agg_scatter_index_2])
    didx = jnp.stack(
        [agg_node_index_0, agg_node_index_1, agg_node_index_2])
    sidx = jnp.pad(sidx, ((0, 0), (0, _EPAD - _E)))
    didx = jnp.pad(didx, ((0, 0), (0, _EPAD - _E)), constant_values=_N)

    cur = _tc_pre(x, W_pre)
    parts = _sc_agg(cur, sidx, didx)
    cur = _tc_layer(cur, parts, mlp_w1[0], mlp_w2[0], eps)
    parts = _sc_agg(cur, sidx, didx)
    return _tc_layer_head(cur, parts, mlp_w1[1], mlp_w2[1], eps,
                          head_w, head_b)
